# baseline (speedup 1.0000x reference)
import functools

import jax
import jax.numpy as jnp
from jax import lax
from jax.experimental import pallas as pl
from jax.experimental.pallas import tpu as pltpu
from jax.experimental.pallas import tpu_sc as plsc

V = 1000000
D = 64
R = 4096
C = 200
NC = 2
NS = 16
NW = NC * NS
IPW = R // NW
NBUF = 4
L = 16

_mesh = plsc.VectorSubcoreMesh(core_axis_name="c", subcore_axis_name="s")


@functools.partial(
    pl.kernel,
    mesh=_mesh,
    out_type=jax.ShapeDtypeStruct((C, D, R), jnp.float32),
    compiler_params=pltpu.CompilerParams(
        use_tc_tiling_on_sc=False, needs_layout_passes=False
    ),
    scratch_types=[
        pltpu.VMEM((IPW, C), jnp.int32),
        pltpu.VMEM((NBUF, IPW), jnp.int32),
        pltpu.VMEM((NBUF, IPW, D), jnp.float32),
        pltpu.VMEM((NBUF, D, IPW), jnp.float32),
        pltpu.SemaphoreType.DMA((NBUF,)),
        pltpu.SemaphoreType.DMA((NBUF,)),
    ],
)
def _gather_kernel(x_hbm, table_hbm, out_hbm, xv, colv, rows_v, tv, gsem, osem):
    wid = lax.axis_index("s") * NC + lax.axis_index("c")
    ibase = wid * IPW

    pltpu.sync_copy(x_hbm.at[pl.ds(ibase, IPW)], xv)

    lane = lax.broadcasted_iota(jnp.int32, (L,), 0)

    def build_col(j, s):
        for q in range(IPW // L):
            vals = plsc.load_gather(xv, [lane + q * L, jnp.full((L,), j, jnp.int32)])
            colv[s, pl.ds(q * L, L)] = vals

    def gather_desc(j, s):
        del j
        return pltpu.make_async_copy(
            table_hbm.at[colv.at[s]], rows_v.at[s], gsem.at[s]
        )

    def out_desc(j, s):
        return pltpu.make_async_copy(
            tv.at[s], out_hbm.at[j, :, pl.ds(ibase, IPW)], osem.at[s]
        )

    def transpose(s):
        for e in range(D):
            e_splat = jnp.full((L,), e, jnp.int32)
            for q in range(IPW // L):
                vals = plsc.load_gather(
                    rows_v.at[s], [lane + q * L, e_splat]
                )
                tv[s, e, pl.ds(q * L, L)] = vals

    for t in range(NBUF):
        build_col(t, t)
        gather_desc(t, t).start()

    def step(j, carry):
        s = jnp.bitwise_and(j, NBUF - 1)
        gather_desc(j, s).wait()

        @pl.when(j >= NBUF)
        def _():
            out_desc(j - NBUF, s).wait()

        transpose(s)
        out_desc(j, s).start()

        @pl.when(j < C - NBUF)
        def _():
            build_col(j + NBUF, s)
            gather_desc(j + NBUF, s).start()

        return carry

    lax.fori_loop(0, C, step, 0)

    for t in range(NBUF):
        out_desc(C - NBUF + t, t).wait()


def kernel(x, weight):
    out_t = _gather_kernel(x.astype(jnp.int32), weight)
    return jnp.transpose(out_t, (2, 0, 1))


# speedup vs baseline: 1.7564x; 1.7564x over previous
import functools

import jax
import jax.numpy as jnp
from jax import lax
from jax.experimental import pallas as pl
from jax.experimental.pallas import tpu as pltpu
from jax.experimental.pallas import tpu_sc as plsc

V = 1000000
D = 64
R = 4096
C = 200
NC = 2
NS = 16
NW = NC * NS
XPW = R // NW
CA = 104
CB = C - CA
NBUF = 4
ROUNDS = XPW // NBUF

_mesh = plsc.VectorSubcoreMesh(core_axis_name="c", subcore_axis_name="s")


@functools.partial(
    pl.kernel,
    mesh=_mesh,
    out_type=jax.ShapeDtypeStruct((R, C, D), jnp.float32),
    compiler_params=pltpu.CompilerParams(use_tc_tiling_on_sc=False),
    scratch_types=[
        pltpu.VMEM((XPW, C), jnp.int32),
        pltpu.VMEM((NBUF, C, D), jnp.float32),
        pltpu.SemaphoreType.DMA((NBUF,)),
    ],
)
def _gather_kernel(x_hbm, table_hbm, out_hbm, idx_v, rows_v, gsem):
    wid = lax.axis_index("s") * NC + lax.axis_index("c")
    xbase = wid * XPW

    pltpu.sync_copy(x_hbm.at[pl.ds(xbase, XPW)], idx_v)

    def gather_descs(j, s):
        a = pltpu.make_async_copy(
            table_hbm.at[idx_v.at[j, pl.ds(0, CA)]],
            rows_v.at[s, pl.ds(0, CA)],
            gsem.at[s],
        )
        b = pltpu.make_async_copy(
            table_hbm.at[idx_v.at[j, pl.ds(CA, CB)]],
            rows_v.at[s, pl.ds(CA, CB)],
            gsem.at[s],
        )
        return a, b

    def start_gathers(j, s):
        a, b = gather_descs(j, s)
        a.start()
        b.start()

    def wait_gathers(j, s):
        a, b = gather_descs(j, s)
        a.wait()
        b.wait()

    def copy_out(j, s):
        pltpu.sync_copy(rows_v.at[s], out_hbm.at[xbase + j])

    for s in range(NBUF):
        start_gathers(s, s)

    def round_body(r, carry):
        for s in range(NBUF):
            j = r * NBUF + s
            wait_gathers(j, s)
            copy_out(j, s)
            start_gathers(j + NBUF, s)
        return carry

    lax.fori_loop(0, ROUNDS - 1, round_body, 0)

    for s in range(NBUF):
        j = (ROUNDS - 1) * NBUF + s
        wait_gathers(j, s)
        copy_out(j, s)


def kernel(x, weight):
    wp = jnp.concatenate(
        [weight, jnp.zeros((V, D), jnp.float32)], axis=1
    ).reshape(2 * V, D)
    return _gather_kernel(x.astype(jnp.int32) * 2, wp)
